# baseline (speedup 1.0000x reference)
import functools

import jax
import jax.numpy as jnp
from jax.experimental import pallas as pl
from jax.experimental.pallas import tpu as pltpu

_NUM_CODES = 1024
_CODE_DIM = 64
_BETA = 0.25


def _vq_kernel(z_ref, cb_ref, zq_ref, idx_ref, counts_ref, loss_ref):
    cb = cb_ref[...]
    e_sq = jnp.sum(cb * cb, axis=1, keepdims=True)
    cbm2 = cb * -2.0
    code_iota = jax.lax.broadcasted_iota(jnp.int32, (_NUM_CODES, 1024), 0)
    iota_f = code_iota.astype(jnp.float32)
    sub_iota = jax.lax.broadcasted_iota(jnp.int32, (8, 1024), 0)
    sub_iota_f = sub_iota.astype(jnp.float32)
    ones_col = jnp.ones((1024, 1), jnp.bfloat16)

    counts = None
    loss = None
    for i in range(z_ref.shape[0]):
        z_b = z_ref[i]
        z_sq = jnp.sum(z_b * z_b, axis=0, keepdims=True)
        s = jax.lax.dot_general(cbm2, z_b, (((1,), (0,)), ((), ())),
                                preferred_element_type=jnp.float32)

        m_val = (z_sq + e_sq[0:8]) + s[0:8]
        m_chunk = jnp.zeros_like(m_val)
        for k in range(1, _NUM_CODES // 8):
            dk = (z_sq + e_sq[8 * k:8 * k + 8]) + s[8 * k:8 * k + 8]
            take = dk < m_val
            m_val = jnp.where(take, dk, m_val)
            m_chunk = jnp.where(take, float(k), m_chunk)

        g = m_chunk * 8.0 + sub_iota_f
        r = 8
        while r > 1:
            h = r // 2
            va, vb = m_val[:h], m_val[h:]
            ga, gb = g[:h], g[h:]
            take = (vb < va) | ((vb == va) & (gb < ga))
            m_val = jnp.where(take, vb, va)
            g = jnp.where(take, gb, ga)
            r = h
        idx_f = g
        idx_ref[i] = idx_f.astype(jnp.int32)

        onehot = (iota_f == idx_f).astype(jnp.bfloat16)
        z_q = jax.lax.dot_general(cb, onehot, (((0,), (0,)), ((), ())),
                                  preferred_element_type=jnp.float32)
        zq_ref[i] = z_q

        c_i = jax.lax.dot_general(onehot, ones_col, (((1,), (0,)), ((), ())),
                                  preferred_element_type=jnp.float32)
        counts = c_i if counts is None else counts + c_i
        loss = m_val if loss is None else loss + m_val

    counts_ref[0] = counts
    loss_ref[0] = loss


def _finalize_kernel(counts_ref, loss_ref, sums_ref, *, total_px):
    loss_all = jnp.sum(loss_ref[...], axis=(0, 1))
    loss_sum = jnp.sum(loss_all, keepdims=True).reshape(1, 1)
    vq_loss = loss_sum * ((1.0 + _BETA) / total_px)
    counts = jnp.sum(counts_ref[...], axis=0)
    p = counts * (1.0 / (total_px / _CODE_DIM))
    ent = jnp.sum(p * jnp.log(p + 1e-10), axis=0, keepdims=True)
    perplexity = jnp.exp(-ent)
    sums_ref[0:1, 0:1] = vq_loss
    sums_ref[0:1, 1:2] = perplexity


def kernel(z, codebook):
    B, C, H, W = z.shape
    P = H * W
    BPB = 8
    nsteps = B // BPB
    z_r = z.reshape(B, C, P)
    total_px = B * C * P

    zq3, idx3, counts_p, loss_p = pl.pallas_call(
        _vq_kernel,
        grid=(nsteps,),
        in_specs=[
            pl.BlockSpec((BPB, C, P), lambda b: (b, 0, 0)),
            pl.BlockSpec((_NUM_CODES, _CODE_DIM), lambda b: (0, 0)),
        ],
        out_specs=[
            pl.BlockSpec((BPB, C, P), lambda b: (b, 0, 0)),
            pl.BlockSpec((BPB, 1, P), lambda b: (b, 0, 0)),
            pl.BlockSpec((1, _NUM_CODES, 1), lambda b: (b, 0, 0)),
            pl.BlockSpec((1, 1, P), lambda b: (b, 0, 0)),
        ],
        out_shape=[
            jax.ShapeDtypeStruct((B, C, P), jnp.float32),
            jax.ShapeDtypeStruct((B, 1, P), jnp.int32),
            jax.ShapeDtypeStruct((nsteps, _NUM_CODES, 1), jnp.float32),
            jax.ShapeDtypeStruct((nsteps, 1, P), jnp.float32),
        ],
        compiler_params=pltpu.CompilerParams(
            dimension_semantics=("parallel",)),
    )(z_r, codebook)

    sums = pl.pallas_call(
        functools.partial(_finalize_kernel, total_px=total_px),
        out_shape=jax.ShapeDtypeStruct((8, 128), jnp.float32),
    )(counts_p, loss_p)

    z_q_st = zq3.reshape(B, C, H, W)
    indices = idx3.reshape(B, H, W)
    return (z_q_st, indices, sums[0, 0], sums[0, 1])


# speedup vs baseline: 1.0788x; 1.0788x over previous
import functools

import jax
import jax.numpy as jnp
from jax.experimental import pallas as pl
from jax.experimental.pallas import tpu as pltpu

_NUM_CODES = 1024
_CODE_DIM = 64
_BETA = 0.25


def _vq_kernel(z_ref, cb_ref, zq_ref, idx_ref, counts_ref, loss_ref):
    cb = cb_ref[...]
    e_sq = jnp.sum(cb * cb, axis=1, keepdims=True)
    cbm2 = cb * -2.0
    code_iota = jax.lax.broadcasted_iota(jnp.int32, (_NUM_CODES, 1024), 0)
    iota_f = code_iota.astype(jnp.float32)
    sub_iota = jax.lax.broadcasted_iota(jnp.int32, (8, 1024), 0)
    sub_iota_f = sub_iota.astype(jnp.float32)
    ones_col = jnp.ones((1024, 1), jnp.bfloat16)

    counts = None
    loss = None
    for i in range(z_ref.shape[0]):
        z_b = z_ref[i]
        z_sq = jnp.sum(z_b * z_b, axis=0, keepdims=True)
        s = jax.lax.dot_general(cbm2, z_b, (((1,), (0,)), ((), ())),
                                preferred_element_type=jnp.float32)

        m_val = (z_sq + e_sq[0:8]) + s[0:8]
        m_chunk = jnp.zeros_like(m_val)
        for k in range(1, _NUM_CODES // 8):
            dk = (z_sq + e_sq[8 * k:8 * k + 8]) + s[8 * k:8 * k + 8]
            take = dk < m_val
            m_val = jnp.where(take, dk, m_val)
            m_chunk = jnp.where(take, float(k), m_chunk)

        g = m_chunk * 8.0 + sub_iota_f
        r = 8
        while r > 1:
            h = r // 2
            va, vb = m_val[:h], m_val[h:]
            ga, gb = g[:h], g[h:]
            take = (vb < va) | ((vb == va) & (gb < ga))
            m_val = jnp.where(take, vb, va)
            g = jnp.where(take, gb, ga)
            r = h
        idx_f = g
        idx_ref[i] = idx_f.astype(jnp.int32)

        onehot = (iota_f == idx_f).astype(jnp.bfloat16)
        z_q = jax.lax.dot_general(cb, onehot, (((0,), (0,)), ((), ())),
                                  preferred_element_type=jnp.float32)
        zq_ref[i] = z_q

        c_i = jnp.sum(onehot.astype(jnp.float32), axis=1, keepdims=True)
        counts = c_i if counts is None else counts + c_i
        loss = m_val if loss is None else loss + m_val

    counts_ref[0] = counts
    loss_ref[0] = loss


def _finalize_kernel(counts_ref, loss_ref, sums_ref, *, total_px):
    loss_all = jnp.sum(loss_ref[...], axis=(0, 1))
    loss_sum = jnp.sum(loss_all, keepdims=True).reshape(1, 1)
    vq_loss = loss_sum * ((1.0 + _BETA) / total_px)
    counts = jnp.sum(counts_ref[...], axis=0)
    p = counts * (1.0 / (total_px / _CODE_DIM))
    ent = jnp.sum(p * jnp.log(p + 1e-10), axis=0, keepdims=True)
    perplexity = jnp.exp(-ent)
    sums_ref[0:1, 0:1] = vq_loss
    sums_ref[0:1, 1:2] = perplexity


def kernel(z, codebook):
    B, C, H, W = z.shape
    P = H * W
    BPB = 8
    nsteps = B // BPB
    z_r = z.reshape(B, C, P)
    total_px = B * C * P

    zq3, idx3, counts_p, loss_p = pl.pallas_call(
        _vq_kernel,
        grid=(nsteps,),
        in_specs=[
            pl.BlockSpec((BPB, C, P), lambda b: (b, 0, 0)),
            pl.BlockSpec((_NUM_CODES, _CODE_DIM), lambda b: (0, 0)),
        ],
        out_specs=[
            pl.BlockSpec((BPB, C, P), lambda b: (b, 0, 0)),
            pl.BlockSpec((BPB, 1, P), lambda b: (b, 0, 0)),
            pl.BlockSpec((1, _NUM_CODES, 1), lambda b: (b, 0, 0)),
            pl.BlockSpec((1, 1, P), lambda b: (b, 0, 0)),
        ],
        out_shape=[
            jax.ShapeDtypeStruct((B, C, P), jnp.float32),
            jax.ShapeDtypeStruct((B, 1, P), jnp.int32),
            jax.ShapeDtypeStruct((nsteps, _NUM_CODES, 1), jnp.float32),
            jax.ShapeDtypeStruct((nsteps, 1, P), jnp.float32),
        ],
        compiler_params=pltpu.CompilerParams(
            dimension_semantics=("parallel",)),
    )(z_r, codebook)

    sums = pl.pallas_call(
        functools.partial(_finalize_kernel, total_px=total_px),
        out_shape=jax.ShapeDtypeStruct((8, 128), jnp.float32),
    )(counts_p, loss_p)

    z_q_st = zq3.reshape(B, C, H, W)
    indices = idx3.reshape(B, H, W)
    return (z_q_st, indices, sums[0, 0], sums[0, 1])


# speedup vs baseline: 1.1273x; 1.0450x over previous
import functools

import jax
import jax.numpy as jnp
from jax.experimental import pallas as pl
from jax.experimental.pallas import tpu as pltpu

_NUM_CODES = 1024
_CODE_DIM = 64
_BETA = 0.25


def _vq_kernel(z_ref, cb_ref, zq_ref, idx_ref, sums_ref,
               counts_ref, loss_ref, *, nsteps, total_px):
    b = pl.program_id(0)
    cb = cb_ref[...]
    e_sq = jnp.sum(cb * cb, axis=1, keepdims=True)
    cbm2 = cb * -2.0
    code_iota = jax.lax.broadcasted_iota(jnp.int32, (_NUM_CODES, 1024), 0)
    iota_f = code_iota.astype(jnp.float32)
    sub_iota = jax.lax.broadcasted_iota(jnp.int32, (8, 1024), 0)
    sub_iota_f = sub_iota.astype(jnp.float32)
    ones_col = jnp.ones((1024, 1), jnp.bfloat16)

    counts = None
    loss = None
    for i in range(z_ref.shape[0]):
        z_b = z_ref[i]
        z_sq = jnp.sum(z_b * z_b, axis=0, keepdims=True)
        s = jax.lax.dot_general(cbm2, z_b, (((1,), (0,)), ((), ())),
                                preferred_element_type=jnp.float32)

        m_val = (z_sq + e_sq[0:8]) + s[0:8]
        m_chunk = jnp.zeros_like(m_val)
        for k in range(1, _NUM_CODES // 8):
            dk = (z_sq + e_sq[8 * k:8 * k + 8]) + s[8 * k:8 * k + 8]
            take = dk < m_val
            m_val = jnp.where(take, dk, m_val)
            m_chunk = jnp.where(take, float(k), m_chunk)

        g = m_chunk * 8.0 + sub_iota_f
        r = 8
        while r > 1:
            h = r // 2
            va, vb = m_val[:h], m_val[h:]
            ga, gb = g[:h], g[h:]
            take = (vb < va) | ((vb == va) & (gb < ga))
            m_val = jnp.where(take, vb, va)
            g = jnp.where(take, gb, ga)
            r = h
        idx_f = g
        idx_ref[i] = idx_f.astype(jnp.int32)

        onehot = (iota_f == idx_f).astype(jnp.bfloat16)
        z_q = jax.lax.dot_general(cb, onehot, (((0,), (0,)), ((), ())),
                                  preferred_element_type=jnp.float32)
        zq_ref[i] = z_q

        c_i = jnp.sum(onehot.astype(jnp.float32), axis=1, keepdims=True)
        counts = c_i if counts is None else counts + c_i
        loss = m_val if loss is None else loss + m_val

    @pl.when(b == 0)
    def _init():
        counts_ref[...] = counts
        loss_ref[...] = loss

    @pl.when(b > 0)
    def _accum():
        counts_ref[...] += counts
        loss_ref[...] += loss

    @pl.when(b == nsteps - 1)
    def _finalize():
        loss_sum = jnp.sum(loss_ref[...], axis=1, keepdims=True)
        vq_loss = loss_sum * ((1.0 + _BETA) / total_px)
        p = counts_ref[...] * (1.0 / (total_px / _CODE_DIM))
        ent = jnp.sum(p * jnp.log(p + 1e-10), axis=0, keepdims=True)
        perplexity = jnp.exp(-ent)
        sums_ref[0:1, 0:1] = vq_loss
        sums_ref[0:1, 1:2] = perplexity


def kernel(z, codebook):
    B, C, H, W = z.shape
    P = H * W
    BPB = 8
    nsteps = B // BPB
    z_r = z.reshape(B, C, P)
    total_px = B * C * P

    zq3, idx3, sums = pl.pallas_call(
        functools.partial(_vq_kernel, nsteps=nsteps, total_px=total_px),
        grid=(nsteps,),
        in_specs=[
            pl.BlockSpec((BPB, C, P), lambda b: (b, 0, 0)),
            pl.BlockSpec((_NUM_CODES, _CODE_DIM), lambda b: (0, 0)),
        ],
        out_specs=[
            pl.BlockSpec((BPB, C, P), lambda b: (b, 0, 0)),
            pl.BlockSpec((BPB, 1, P), lambda b: (b, 0, 0)),
            pl.BlockSpec((8, 128), lambda b: (0, 0)),
        ],
        out_shape=[
            jax.ShapeDtypeStruct((B, C, P), jnp.float32),
            jax.ShapeDtypeStruct((B, 1, P), jnp.int32),
            jax.ShapeDtypeStruct((8, 128), jnp.float32),
        ],
        scratch_shapes=[
            pltpu.VMEM((_NUM_CODES, 1), jnp.float32),
            pltpu.VMEM((1, P), jnp.float32),
        ],
    )(z_r, codebook)

    z_q_st = zq3.reshape(B, C, H, W)
    indices = idx3.reshape(B, H, W)
    return (z_q_st, indices, sums[0, 0], sums[0, 1])


# speedup vs baseline: 1.1278x; 1.0004x over previous
import functools

import jax
import jax.numpy as jnp
from jax.experimental import pallas as pl
from jax.experimental.pallas import tpu as pltpu

_NUM_CODES = 1024
_CODE_DIM = 64
_BETA = 0.25


def _vq_kernel(z_ref, cb_ref, zq_ref, idx_ref, sums_ref,
               counts_ref, loss_ref, *, nsteps, total_px):
    b = pl.program_id(0)
    cb = cb_ref[...]
    e_sq = jnp.sum(cb * cb, axis=1, keepdims=True)
    cbm2 = cb * -2.0
    code_iota = jax.lax.broadcasted_iota(jnp.int32, (_NUM_CODES, 1024), 0)
    iota_f = code_iota.astype(jnp.float32)
    sub_iota = jax.lax.broadcasted_iota(jnp.int32, (8, 1024), 0)
    sub_iota_f = sub_iota.astype(jnp.float32)

    counts = None
    loss = None
    for i in range(z_ref.shape[0]):
        z_b = z_ref[i]
        z_sq = jnp.sum(z_b * z_b, axis=0, keepdims=True)
        s = jax.lax.dot_general(cbm2, z_b, (((1,), (0,)), ((), ())),
                                preferred_element_type=jnp.float32)

        m_val = (z_sq + e_sq[0:8]) + s[0:8]
        m_chunk = jnp.zeros_like(m_val)
        for k in range(1, _NUM_CODES // 8):
            dk = (z_sq + e_sq[8 * k:8 * k + 8]) + s[8 * k:8 * k + 8]
            take = dk < m_val
            m_val = jnp.where(take, dk, m_val)
            m_chunk = jnp.where(take, float(k), m_chunk)

        g = m_chunk * 8.0 + sub_iota_f
        r = 8
        while r > 1:
            h = r // 2
            va, vb = m_val[:h], m_val[h:]
            ga, gb = g[:h], g[h:]
            take = (vb < va) | ((vb == va) & (gb < ga))
            m_val = jnp.where(take, vb, va)
            g = jnp.where(take, gb, ga)
            r = h
        idx_f = g
        idx_ref[i] = idx_f.astype(jnp.int32)

        onehot = (iota_f == idx_f).astype(jnp.bfloat16)
        z_q = jax.lax.dot_general(cb, onehot, (((0,), (0,)), ((), ())),
                                  preferred_element_type=jnp.float32)
        zq_ref[i] = z_q

        c_i = jnp.sum(onehot.astype(jnp.float32), axis=1, keepdims=True)
        counts = c_i if counts is None else counts + c_i
        loss = m_val if loss is None else loss + m_val

    @pl.when(b == 0)
    def _init():
        counts_ref[...] = counts
        loss_ref[...] = loss

    @pl.when(b > 0)
    def _accum():
        counts_ref[...] += counts
        loss_ref[...] += loss

    @pl.when(b == nsteps - 1)
    def _finalize():
        loss_sum = jnp.sum(loss_ref[...], axis=1, keepdims=True)
        vq_loss = loss_sum * ((1.0 + _BETA) / total_px)
        p = counts_ref[...] * (1.0 / (total_px / _CODE_DIM))
        ent = jnp.sum(p * jnp.log(p + 1e-10), axis=0, keepdims=True)
        perplexity = jnp.exp(-ent)
        sums_ref[0:1, 0:1] = vq_loss
        sums_ref[0:1, 1:2] = perplexity


def kernel(z, codebook):
    B, C, H, W = z.shape
    P = H * W
    BPB = 8
    nsteps = B // BPB
    z_r = z.reshape(B, C, P)
    total_px = B * C * P

    zq3, idx3, sums = pl.pallas_call(
        functools.partial(_vq_kernel, nsteps=nsteps, total_px=total_px),
        grid=(nsteps,),
        in_specs=[
            pl.BlockSpec((BPB, C, P), lambda b: (b, 0, 0)),
            pl.BlockSpec((_NUM_CODES, _CODE_DIM), lambda b: (0, 0)),
        ],
        out_specs=[
            pl.BlockSpec((BPB, C, P), lambda b: (b, 0, 0)),
            pl.BlockSpec((BPB, 1, P), lambda b: (b, 0, 0)),
            pl.BlockSpec((8, 128), lambda b: (0, 0)),
        ],
        out_shape=[
            jax.ShapeDtypeStruct((B, C, P), jnp.float32),
            jax.ShapeDtypeStruct((B, 1, P), jnp.int32),
            jax.ShapeDtypeStruct((8, 128), jnp.float32),
        ],
        scratch_shapes=[
            pltpu.VMEM((_NUM_CODES, 1), jnp.float32),
            pltpu.VMEM((1, P), jnp.float32),
        ],
    )(z_r, codebook)

    z_q_st = zq3.reshape(B, C, H, W)
    indices = idx3.reshape(B, H, W)
    return (z_q_st, indices, sums[0, 0], sums[0, 1])
